# baseline (speedup 1.0000x reference)
import functools

import jax
import jax.numpy as jnp
from jax import lax
from jax.experimental import pallas as pl
from jax.experimental.pallas import tpu as pltpu
from jax.experimental.pallas import tpu_sc as plsc

N_ROWS = 8192
C_DIM = 256
K_CODES = 8192
ROW_TILE = 256
K_CHUNK = 1024
N_TILES = N_ROWS // ROW_TILE
N_CHUNKS = K_CODES // K_CHUNK


def _argmin_body(x_ref, cb_ref, idx_ref, mind2_ref, d2_s, cbsq_s, iota_s,
                 xsqb_s, hb_s):
    @pl.when(pl.program_id(0) == 0)
    def _init():
        for c in range(N_CHUNKS):
            cbc = cb_ref[pl.ds(c * K_CHUNK, K_CHUNK), :]
            s = jnp.sum(cbc * cbc, axis=1)
            cbsq_s[0, pl.ds(c * K_CHUNK, K_CHUNK)] = s
        iota_s[0, :] = lax.broadcasted_iota(jnp.int32, (K_CODES,), 0)

    xt = x_ref[...]
    x_sq = jnp.sum(xt * xt, axis=1, keepdims=True)
    xt2 = xt + xt
    xsqb_s[...] = jnp.broadcast_to(x_sq, (ROW_TILE, K_CHUNK))

    def sweep1(c, run):
        cb = cb_ref[pl.ds(c * K_CHUNK, K_CHUNK), :]
        cross2 = lax.dot_general(
            xt2, cb, (((1,), (1,)), ((), ())),
            preferred_element_type=jnp.float32)
        cbsq = cbsq_s[0:1, pl.ds(c * K_CHUNK, K_CHUNK)]
        d2 = (xsqb_s[...] - cross2) + cbsq
        d2_s[:, pl.ds(c * K_CHUNK, K_CHUNK)] = d2
        m = jnp.min(d2.reshape(ROW_TILE, K_CHUNK // 128, 128), axis=1)
        return jnp.minimum(run, m)

    run0 = jnp.full((ROW_TILE, 128), jnp.inf, jnp.float32)
    md2 = jnp.min(lax.fori_loop(0, N_CHUNKS, sweep1, run0),
                  axis=1, keepdims=True)

    y = jnp.maximum(md2, 0.0)
    m = jnp.sqrt(y)
    for _ in range(6):
        y_next = lax.bitcast_convert_type(
            lax.bitcast_convert_type(y, jnp.int32) + 1, jnp.float32)
        ok = jnp.sqrt(y_next) <= m
        y = jnp.where(ok, y_next, y)

    hb_s[...] = jnp.broadcast_to(y, (ROW_TILE, K_CHUNK))

    def sweep2(c, run):
        d2 = d2_s[:, pl.ds(c * K_CHUNK, K_CHUNK)]
        ki = iota_s[0:1, pl.ds(c * K_CHUNK, K_CHUNK)]
        enc = jnp.where(d2 <= hb_s[...],
                        jnp.broadcast_to(ki, d2.shape), K_CODES)
        e = jnp.min(enc.reshape(ROW_TILE, K_CHUNK // 128, 128), axis=1)
        return jnp.minimum(run, e)

    run0i = jnp.full((ROW_TILE, 128), K_CODES, jnp.int32)
    best_i = jnp.min(lax.fori_loop(0, N_CHUNKS, sweep2, run0i), axis=1)
    idx_ref[0, 0, :] = best_i
    mind2_ref[0, 0, :] = (m * m)[:, 0]


def _argmin_call(xp, codebook):
    return pl.pallas_call(
        _argmin_body,
        grid=(N_TILES,),
        in_specs=[
            pl.BlockSpec((ROW_TILE, C_DIM), lambda i: (i, 0)),
            pl.BlockSpec((K_CODES, C_DIM), lambda i: (0, 0)),
        ],
        out_specs=[
            pl.BlockSpec((1, 1, ROW_TILE), lambda i: (i, 0, 0)),
            pl.BlockSpec((1, 1, ROW_TILE), lambda i: (i, 0, 0)),
        ],
        out_shape=[
            jax.ShapeDtypeStruct((N_TILES, 1, ROW_TILE), jnp.int32),
            jax.ShapeDtypeStruct((N_TILES, 1, ROW_TILE), jnp.float32),
        ],
        scratch_shapes=[
            pltpu.VMEM((ROW_TILE, K_CODES), jnp.float32),
            pltpu.VMEM((1, K_CODES), jnp.float32),
            pltpu.VMEM((1, K_CODES), jnp.int32),
            pltpu.VMEM((ROW_TILE, K_CHUNK), jnp.float32),
            pltpu.VMEM((ROW_TILE, K_CHUNK), jnp.float32),
        ],
    )(xp, codebook)


def _make_sc_gather():
    info = plsc.get_sparse_core_info()
    nw = info.num_cores * info.num_subcores
    b_per_w = N_ROWS // nw
    mesh = plsc.VectorSubcoreMesh(core_axis_name="c", subcore_axis_name="s")

    @functools.partial(
        pl.kernel, mesh=mesh,
        out_type=jax.ShapeDtypeStruct((N_ROWS, C_DIM), jnp.float32),
        scratch_types=[
            pltpu.VMEM((b_per_w,), jnp.int32),
            pltpu.VMEM((b_per_w, C_DIM), jnp.float32),
            pltpu.SemaphoreType.DMA,
        ],
    )
    def gather(table_hbm, idx_hbm, out_hbm, idx_v, rows_v, sem):
        wid = lax.axis_index("s") * info.num_cores + lax.axis_index("c")
        base = wid * b_per_w
        pltpu.sync_copy(idx_hbm.at[pl.ds(base, b_per_w)], idx_v)
        pltpu.async_copy(table_hbm.at[idx_v], rows_v, sem).wait()
        pltpu.sync_copy(rows_v, out_hbm.at[pl.ds(base, b_per_w)])

    return gather


_sc_gather = None


def kernel(x, codebook):
    global _sc_gather
    if _sc_gather is None:
        _sc_gather = _make_sc_gather()
    B, C, L = x.shape
    xp = jnp.transpose(x, (0, 2, 1)).reshape(N_ROWS, C_DIM)
    idx3, mind2 = _argmin_call(xp, codebook)
    idx_flat = idx3.reshape(N_ROWS)
    quant = _sc_gather(codebook, idx_flat)
    loss = jnp.sum(mind2) / (N_ROWS * C_DIM)
    quant_st = jnp.transpose(quant.reshape(B, L, C), (0, 2, 1))
    return quant_st, loss, loss, idx_flat.reshape(B, L)


# speedup vs baseline: 2.3866x; 2.3866x over previous
import functools

import jax
import jax.numpy as jnp
from jax import lax
from jax.experimental import pallas as pl
from jax.experimental.pallas import tpu as pltpu
from jax.experimental.pallas import tpu_sc as plsc

N_ROWS = 8192
C_DIM = 256
K_CODES = 8192
ROW_TILE = 256
K_CHUNK = 1024
N_TILES = N_ROWS // ROW_TILE
N_CHUNKS = K_CODES // K_CHUNK


def _argmin_body(x_ref, cb_ref, idx_ref, mind2_ref, cbsq_s, iota_s, xsqb_s):
    @pl.when(pl.program_id(0) == 0)
    def _init():
        for c in range(N_CHUNKS):
            cbc = cb_ref[pl.ds(c * K_CHUNK, K_CHUNK), :]
            s = jnp.sum(cbc * cbc, axis=1)
            cbsq_s[0, pl.ds(c * K_CHUNK, K_CHUNK)] = s
        iota_s[0, :] = lax.broadcasted_iota(jnp.int32, (K_CODES,), 0)

    xt = x_ref[...]
    x_sq = jnp.sum(xt * xt, axis=1, keepdims=True)
    xt2 = xt + xt
    xsqb_s[...] = jnp.broadcast_to(x_sq, (ROW_TILE, K_CHUNK))

    run_d = jnp.full((ROW_TILE, 128), jnp.inf, jnp.float32)
    run_i = jnp.full((ROW_TILE, 128), K_CODES, jnp.int32)
    for c in range(N_CHUNKS):
        cb = cb_ref[c * K_CHUNK:(c + 1) * K_CHUNK, :]
        cross2 = lax.dot_general(
            xt2, cb, (((1,), (1,)), ((), ())),
            preferred_element_type=jnp.float32)
        cbsq = cbsq_s[0:1, c * K_CHUNK:(c + 1) * K_CHUNK]
        d2 = (xsqb_s[...] - cross2) + cbsq
        dist = jnp.sqrt(jnp.maximum(d2, 0.0))
        for j in range(K_CHUNK // 128):
            dj = dist[:, j * 128:(j + 1) * 128]
            kj = jnp.broadcast_to(
                iota_s[0:1, c * K_CHUNK + j * 128:c * K_CHUNK + (j + 1) * 128],
                (ROW_TILE, 128))
            take = dj < run_d
            run_d = jnp.where(take, dj, run_d)
            run_i = jnp.where(take, kj, run_i)

    loc = jnp.min(run_d, axis=1, keepdims=True)
    best_i = jnp.min(jnp.where(run_d == loc, run_i, K_CODES), axis=1)
    idx_ref[0, 0, :] = best_i
    mind2_ref[0, 0, :] = (loc * loc)[:, 0]


def _argmin_call(xp, codebook):
    return pl.pallas_call(
        _argmin_body,
        grid=(N_TILES,),
        in_specs=[
            pl.BlockSpec((ROW_TILE, C_DIM), lambda i: (i, 0)),
            pl.BlockSpec((K_CODES, C_DIM), lambda i: (0, 0)),
        ],
        out_specs=[
            pl.BlockSpec((1, 1, ROW_TILE), lambda i: (i, 0, 0)),
            pl.BlockSpec((1, 1, ROW_TILE), lambda i: (i, 0, 0)),
        ],
        out_shape=[
            jax.ShapeDtypeStruct((N_TILES, 1, ROW_TILE), jnp.int32),
            jax.ShapeDtypeStruct((N_TILES, 1, ROW_TILE), jnp.float32),
        ],
        scratch_shapes=[
            pltpu.VMEM((1, K_CODES), jnp.float32),
            pltpu.VMEM((1, K_CODES), jnp.int32),
            pltpu.VMEM((ROW_TILE, K_CHUNK), jnp.float32),
        ],
    )(xp, codebook)


def _make_sc_gather():
    info = plsc.get_sparse_core_info()
    nw = info.num_cores * info.num_subcores
    b_per_w = N_ROWS // nw
    mesh = plsc.VectorSubcoreMesh(core_axis_name="c", subcore_axis_name="s")

    @functools.partial(
        pl.kernel, mesh=mesh,
        out_type=jax.ShapeDtypeStruct((N_ROWS, C_DIM), jnp.float32),
        scratch_types=[
            pltpu.VMEM((b_per_w,), jnp.int32),
            pltpu.VMEM((b_per_w, C_DIM), jnp.float32),
            pltpu.SemaphoreType.DMA,
        ],
    )
    def gather(table_hbm, idx_hbm, out_hbm, idx_v, rows_v, sem):
        wid = lax.axis_index("s") * info.num_cores + lax.axis_index("c")
        base = wid * b_per_w
        pltpu.sync_copy(idx_hbm.at[pl.ds(base, b_per_w)], idx_v)
        pltpu.async_copy(table_hbm.at[idx_v], rows_v, sem).wait()
        pltpu.sync_copy(rows_v, out_hbm.at[pl.ds(base, b_per_w)])

    return gather


_sc_gather = None


def kernel(x, codebook):
    global _sc_gather
    if _sc_gather is None:
        _sc_gather = _make_sc_gather()
    B, C, L = x.shape
    xp = jnp.transpose(x, (0, 2, 1)).reshape(N_ROWS, C_DIM)
    idx3, mind2 = _argmin_call(xp, codebook)
    idx_flat = idx3.reshape(N_ROWS)
    quant = _sc_gather(codebook, idx_flat)
    loss = jnp.sum(mind2) / (N_ROWS * C_DIM)
    quant_st = jnp.transpose(quant.reshape(B, L, C), (0, 2, 1))
    return quant_st, loss, loss, idx_flat.reshape(B, L)
